# baseline (speedup 1.0000x reference)
import dataclasses

import jax
import jax.numpy as jnp
from jax import lax
from jax.experimental import pallas as pl
from jax.experimental.pallas import tpu as pltpu
from jax.experimental.pallas import tpu_sc as plsc

_B = 4096
_D = 2048
_NREG = 64
_VR = 256

_BM = 512
_G = _B // _BM

_NC = 2
_NS = 16
_NW = _NC * _NS
_RPW = _B // _NW
_L = 16
_GRP = _RPW // _L


def _tc_body(x_ref, regs_ref, wr_ref, br_ref, ww_ref, bw_ref, wv_ref, bv_ref,
             emb_ref, vm_ref,
             ro_ref, wo_ref, vo_ref, widx_ref, wval_ref, rv_ref, fb_ref,
             ehi_ref, elo_ref):
    x = x_ref[...]

    @pl.when(pl.program_id(0) == 0)
    def _():
        e = vm_ref[0, 0] * emb_ref[...]
        hi = e.astype(jnp.bfloat16)
        ehi_ref[...] = hi
        elo_ref[...] = (e - hi.astype(jnp.float32)).astype(jnp.bfloat16)

    def head(w_ref, b_ref):
        return jnp.dot(x, w_ref[...], preferred_element_type=jnp.float32) \
            + b_ref[...]

    def amax(l):
        k = l.shape[1]
        m = jnp.max(l, axis=-1, keepdims=True)
        indb = l == m
        ind = indb.astype(jnp.bfloat16)
        rowi = lax.broadcasted_iota(jnp.int32, (k, k), 0)
        coli = lax.broadcasted_iota(jnp.int32, (k, k), 1)
        lt = (rowi <= coli).astype(jnp.bfloat16)
        c1 = jnp.dot(ind, lt, preferred_element_type=jnp.float32)
        first = jnp.where(indb & (c1 == 1.0), 1.0, 0.0).astype(jnp.bfloat16)
        iota_col = lax.broadcasted_iota(
            jnp.int32, (k, 1), 0).astype(jnp.bfloat16)
        idxf = jnp.dot(first, iota_col, preferred_element_type=jnp.float32)
        return idxf.astype(jnp.int32)

    rl = head(wr_ref, br_ref)
    wl = head(ww_ref, bw_ref)
    vl = head(wv_ref, bv_ref)
    ro_ref[...] = rl
    wo_ref[...] = wl
    vo_ref[...] = vl
    ridx = amax(rl)
    widx_ref[...] = amax(wl)
    wval_ref[...] = amax(vl)

    regs = regs_ref[...]
    col = lax.broadcasted_iota(jnp.int32, regs.shape, 1)
    sel = jnp.where(col == ridx, regs, 0).astype(jnp.bfloat16)
    ones_col = jnp.ones((_NREG, 1), jnp.bfloat16)
    rvf = jnp.dot(sel, ones_col, preferred_element_type=jnp.float32)
    rv = rvf.astype(jnp.int32)
    rv_ref[...] = rv

    rvc = jnp.minimum(jnp.maximum(rv, 0), _VR - 1)
    vcol = lax.broadcasted_iota(jnp.int32, (rv.shape[0], _VR), 1)
    onehot = (vcol == rvc).astype(jnp.bfloat16)
    fb_ref[...] = (
        jnp.dot(onehot, ehi_ref[...], preferred_element_type=jnp.float32)
        + jnp.dot(onehot, elo_ref[...], preferred_element_type=jnp.float32))


def _tc_call(x, registers, w_r, b_r, w_w, b_w, w_v, b_v, emb, vm):
    f32 = jnp.float32
    i32 = jnp.int32
    in_specs = [
        pl.BlockSpec((_BM, _D), lambda i: (i, 0)),
        pl.BlockSpec((_BM, _NREG), lambda i: (i, 0)),
        pl.BlockSpec((_D, _NREG + 1), lambda i: (0, 0)),
        pl.BlockSpec((1, _NREG + 1), lambda i: (0, 0)),
        pl.BlockSpec((_D, _NREG + 1), lambda i: (0, 0)),
        pl.BlockSpec((1, _NREG + 1), lambda i: (0, 0)),
        pl.BlockSpec((_D, _VR), lambda i: (0, 0)),
        pl.BlockSpec((1, _VR), lambda i: (0, 0)),
        pl.BlockSpec((_VR, _D), lambda i: (0, 0)),
        pl.BlockSpec((1, 1), lambda i: (0, 0)),
    ]
    out_specs = [
        pl.BlockSpec((_BM, _NREG + 1), lambda i: (i, 0)),
        pl.BlockSpec((_BM, _NREG + 1), lambda i: (i, 0)),
        pl.BlockSpec((_BM, _VR), lambda i: (i, 0)),
        pl.BlockSpec((_BM, 1), lambda i: (i, 0)),
        pl.BlockSpec((_BM, 1), lambda i: (i, 0)),
        pl.BlockSpec((_BM, 1), lambda i: (i, 0)),
        pl.BlockSpec((_BM, _D), lambda i: (i, 0)),
    ]
    out_shape = [
        jax.ShapeDtypeStruct((_B, _NREG + 1), f32),
        jax.ShapeDtypeStruct((_B, _NREG + 1), f32),
        jax.ShapeDtypeStruct((_B, _VR), f32),
        jax.ShapeDtypeStruct((_B, 1), i32),
        jax.ShapeDtypeStruct((_B, 1), i32),
        jax.ShapeDtypeStruct((_B, 1), i32),
        jax.ShapeDtypeStruct((_B, _D), f32),
    ]
    return pl.pallas_call(
        _tc_body,
        grid=(_G,),
        in_specs=in_specs,
        out_specs=out_specs,
        out_shape=out_shape,
        scratch_shapes=[
            pltpu.VMEM((_VR, _D), jnp.bfloat16),
            pltpu.VMEM((_VR, _D), jnp.bfloat16),
        ],
        compiler_params=pltpu.CompilerParams(
            dimension_semantics=("arbitrary",)),
    )(x, registers, w_r, b_r, w_w, b_w, w_v, b_v, emb, vm)


def _sc_body(regs_hbm, widx_hbm, wval_hbm, nregs_hbm,
             widx_v, wval_v, regs_v, sem_idx, sem_regs):
    wid = lax.axis_index("s") * _NC + lax.axis_index("c")
    base = wid * _RPW
    c_wi = pltpu.async_copy(widx_hbm.at[pl.ds(base, _RPW)], widx_v, sem_idx)
    c_wv = pltpu.async_copy(wval_hbm.at[pl.ds(base, _RPW)], wval_v, sem_idx)
    c_rg = pltpu.async_copy(regs_hbm.at[pl.ds(base, _RPW)], regs_v, sem_regs)
    c_wi.wait()
    c_wv.wait()
    c_rg.wait()

    for g in range(_GRP):
        sl = pl.ds(g * _L, _L)
        wi = widx_v[sl]
        wv = wval_v[sl]
        rows16 = lax.iota(jnp.int32, _L) + (g * _L)
        wmask = wi < _NREG
        wcol = jnp.minimum(wi, _NREG - 1)
        plsc.store_scatter(regs_v, [rows16, wcol], wv, mask=wmask)

    pltpu.sync_copy(regs_v, nregs_hbm.at[pl.ds(base, _RPW)])


def _sc_call(registers, widx, wval):
    i32 = jnp.int32
    mesh = plsc.VectorSubcoreMesh(core_axis_name="c", subcore_axis_name="s")
    cp = pltpu.CompilerParams()
    if "needs_layout_passes" in pltpu.CompilerParams.__dataclass_fields__:
        cp = dataclasses.replace(cp, needs_layout_passes=False)
    kern = pl.kernel(
        _sc_body,
        out_type=jax.ShapeDtypeStruct((_B, _NREG), i32),
        mesh=mesh,
        scratch_types=[
            pltpu.VMEM((_RPW,), i32),
            pltpu.VMEM((_RPW,), i32),
            pltpu.VMEM((_RPW, _NREG), i32),
            pltpu.SemaphoreType.DMA,
            pltpu.SemaphoreType.DMA,
        ],
        compiler_params=cp,
    )
    return kern(registers, widx, wval)


def kernel(x, registers, W_read, b_read, W_write, b_write, W_val, b_val,
           value_emb, value_mix):
    br = b_read.reshape(1, _NREG + 1)
    bw = b_write.reshape(1, _NREG + 1)
    bv = b_val.reshape(1, _VR)
    vm = value_mix.reshape(1, 1)
    ro, wo, vo, widx, wval, rv, fb = _tc_call(
        x, registers, W_read, br, W_write, bw, W_val, bv, value_emb, vm)
    nregs = _sc_call(registers, widx.reshape(_B), wval.reshape(_B))
    return (ro, wo, vo, nregs, rv.reshape(_B), fb)


# speedup vs baseline: 1.2691x; 1.2691x over previous
import dataclasses

import jax
import jax.numpy as jnp
from jax import lax
from jax.experimental import pallas as pl
from jax.experimental.pallas import tpu as pltpu
from jax.experimental.pallas import tpu_sc as plsc

_B = 4096
_D = 2048
_NREG = 64
_VR = 256

_BM = 512
_G = _B // _BM

_NC = 2
_NS = 16
_NW = _NC * _NS
_RPW = _B // _NW
_L = 16
_GRP = _RPW // _L


def _tc_body(x_ref, regs_ref, wrt_ref, br_ref, wwt_ref, bw_ref, wv_ref,
             bv_ref, emb_ref, vm_ref,
             ro_ref, wo_ref, vo_ref, widx_ref, wval_ref, rv_ref, fb_ref,
             ehi_ref, elo_ref):
    x = x_ref[...]

    @pl.when(pl.program_id(0) == 0)
    def _():
        e = vm_ref[0, 0] * emb_ref[...]
        hi = e.astype(jnp.bfloat16)
        ehi_ref[...] = hi
        elo_ref[...] = (e - hi.astype(jnp.float32)).astype(jnp.bfloat16)

    def amax(l):
        m = jnp.max(l, axis=-1, keepdims=True)
        ii = lax.broadcasted_iota(jnp.int32, l.shape, 1)
        return jnp.min(jnp.where(l == m, ii, l.shape[1]), axis=-1,
                       keepdims=True).astype(jnp.int32)

    rl = jnp.dot(x, wrt_ref[...].T, preferred_element_type=jnp.float32) \
        + br_ref[...]
    wl = jnp.dot(x, wwt_ref[...].T, preferred_element_type=jnp.float32) \
        + bw_ref[...]
    vl = jnp.dot(x, wv_ref[...], preferred_element_type=jnp.float32) \
        + bv_ref[...]
    ro_ref[...] = rl
    wo_ref[...] = wl
    vo_ref[...] = vl
    ridx = amax(rl)
    widx_ref[...] = amax(wl)
    wval_ref[...] = amax(vl)

    regs = regs_ref[...]
    col = lax.broadcasted_iota(jnp.int32, regs.shape, 1)
    sel = jnp.where(col == ridx, regs, 0).astype(jnp.bfloat16)
    ones_col = jnp.ones((_NREG, 1), jnp.bfloat16)
    rvf = jnp.dot(sel, ones_col, preferred_element_type=jnp.float32)
    rv = rvf.astype(jnp.int32)
    rv_ref[...] = rv

    rvc = jnp.minimum(jnp.maximum(rv, 0), _VR - 1)
    vcol = lax.broadcasted_iota(jnp.int32, (rv.shape[0], _VR), 1)
    onehot = (vcol == rvc).astype(jnp.bfloat16)
    fb_ref[...] = (
        jnp.dot(onehot, ehi_ref[...], preferred_element_type=jnp.float32)
        + jnp.dot(onehot, elo_ref[...], preferred_element_type=jnp.float32))


def _tc_call(x, registers, w_r_t, b_r, w_w_t, b_w, w_v, b_v, emb, vm):
    f32 = jnp.float32
    i32 = jnp.int32
    in_specs = [
        pl.BlockSpec((_BM, _D), lambda i: (i, 0)),
        pl.BlockSpec((_BM, _NREG), lambda i: (i, 0)),
        pl.BlockSpec((_NREG + 1, _D), lambda i: (0, 0)),
        pl.BlockSpec((1, _NREG + 1), lambda i: (0, 0)),
        pl.BlockSpec((_NREG + 1, _D), lambda i: (0, 0)),
        pl.BlockSpec((1, _NREG + 1), lambda i: (0, 0)),
        pl.BlockSpec((_D, _VR), lambda i: (0, 0)),
        pl.BlockSpec((1, _VR), lambda i: (0, 0)),
        pl.BlockSpec((_VR, _D), lambda i: (0, 0)),
        pl.BlockSpec((1, 1), lambda i: (0, 0)),
    ]
    out_specs = [
        pl.BlockSpec((_BM, _NREG + 1), lambda i: (i, 0)),
        pl.BlockSpec((_BM, _NREG + 1), lambda i: (i, 0)),
        pl.BlockSpec((_BM, _VR), lambda i: (i, 0)),
        pl.BlockSpec((_BM, 1), lambda i: (i, 0)),
        pl.BlockSpec((_BM, 1), lambda i: (i, 0)),
        pl.BlockSpec((_BM, 1), lambda i: (i, 0)),
        pl.BlockSpec((_BM, _D), lambda i: (i, 0)),
    ]
    out_shape = [
        jax.ShapeDtypeStruct((_B, _NREG + 1), f32),
        jax.ShapeDtypeStruct((_B, _NREG + 1), f32),
        jax.ShapeDtypeStruct((_B, _VR), f32),
        jax.ShapeDtypeStruct((_B, 1), i32),
        jax.ShapeDtypeStruct((_B, 1), i32),
        jax.ShapeDtypeStruct((_B, 1), i32),
        jax.ShapeDtypeStruct((_B, _D), f32),
    ]
    return pl.pallas_call(
        _tc_body,
        grid=(_G,),
        in_specs=in_specs,
        out_specs=out_specs,
        out_shape=out_shape,
        scratch_shapes=[
            pltpu.VMEM((_VR, _D), jnp.bfloat16),
            pltpu.VMEM((_VR, _D), jnp.bfloat16),
        ],
        compiler_params=pltpu.CompilerParams(
            dimension_semantics=("arbitrary",)),
    )(x, registers, w_r_t, b_r, w_w_t, b_w, w_v, b_v, emb, vm)


def _sc_body(regsT_hbm, widx_hbm, wval_hbm, rv_hbm,
             nregsT_hbm, rvflat_hbm,
             widx_v, wval_v, rv_v, regsT_v, rvf_v, sem_idx, sem_regs):
    wid = lax.axis_index("s") * _NC + lax.axis_index("c")
    base = wid * _RPW
    c_wi = pltpu.async_copy(widx_hbm.at[pl.ds(base, _RPW)], widx_v, sem_idx)
    c_wv = pltpu.async_copy(wval_hbm.at[pl.ds(base, _RPW)], wval_v, sem_idx)
    c_rv = pltpu.async_copy(rv_hbm.at[pl.ds(base, _RPW)], rv_v, sem_idx)
    c_rg = pltpu.async_copy(
        regsT_hbm.at[:, pl.ds(base, _RPW)], regsT_v, sem_regs)
    c_wi.wait()
    c_wv.wait()
    c_rv.wait()
    c_rg.wait()

    zeros16 = jnp.zeros((_L,), jnp.int32)
    for g in range(_GRP):
        rows16 = lax.iota(jnp.int32, _L) + (g * _L)
        wi = plsc.load_gather(widx_v, [rows16, zeros16])
        wv = plsc.load_gather(wval_v, [rows16, zeros16])
        rvg = plsc.load_gather(rv_v, [rows16, zeros16])
        rvf_v[pl.ds(g * _L, _L)] = rvg
        wmask = wi < _NREG
        wcol = jnp.minimum(wi, _NREG - 1)
        plsc.store_scatter(regsT_v, [wcol, rows16], wv, mask=wmask)

    pltpu.sync_copy(regsT_v, nregsT_hbm.at[:, pl.ds(base, _RPW)])
    pltpu.sync_copy(rvf_v, rvflat_hbm.at[pl.ds(base, _RPW)])


def _sc_call(regsT, widx, wval, rv):
    i32 = jnp.int32
    mesh = plsc.VectorSubcoreMesh(core_axis_name="c", subcore_axis_name="s")
    cp = pltpu.CompilerParams()
    if "needs_layout_passes" in pltpu.CompilerParams.__dataclass_fields__:
        cp = dataclasses.replace(cp, needs_layout_passes=False)
    kern = pl.kernel(
        _sc_body,
        out_type=[
            jax.ShapeDtypeStruct((_NREG, _B), i32),
            jax.ShapeDtypeStruct((_B,), i32),
        ],
        mesh=mesh,
        scratch_types=[
            pltpu.VMEM((_RPW, 1), i32),
            pltpu.VMEM((_RPW, 1), i32),
            pltpu.VMEM((_RPW, 1), i32),
            pltpu.VMEM((_NREG, _RPW), i32),
            pltpu.VMEM((_RPW,), i32),
            pltpu.SemaphoreType.DMA,
            pltpu.SemaphoreType.DMA,
        ],
        compiler_params=cp,
    )
    return kern(regsT, widx, wval, rv)


def kernel(x, registers, W_read, b_read, W_write, b_write, W_val, b_val,
           value_emb, value_mix):
    br = b_read.reshape(1, _NREG + 1)
    bw = b_write.reshape(1, _NREG + 1)
    bv = b_val.reshape(1, _VR)
    vm = value_mix.reshape(1, 1)
    ro, wo, vo, widx, wval, rv, fb = _tc_call(
        x, registers, W_read.T, br, W_write.T, bw, W_val, bv, value_emb, vm)
    nregsT, rvflat = _sc_call(registers.T, widx, wval, rv)
    return (ro, wo, vo, nregsT.T, rvflat, fb)


# speedup vs baseline: 1.3112x; 1.0332x over previous
import dataclasses

import jax
import jax.numpy as jnp
from jax import lax
from jax.experimental import pallas as pl
from jax.experimental.pallas import tpu as pltpu
from jax.experimental.pallas import tpu_sc as plsc

_B = 4096
_D = 2048
_NREG = 64
_VR = 256

_BM = 512
_G = _B // _BM

_NC = 2
_NS = 16
_NW = _NC * _NS
_RPW = _B // _NW
_L = 16
_GRP = _RPW // _L


def _tc_body(x_ref, regsT_ref, wrt_ref, br_ref, wwt_ref, bw_ref, wv_ref,
             bv_ref, emb_ref, vm_ref,
             ro_ref, wo_ref, vo_ref, widx_ref, wval_ref, rv_ref, fb_ref,
             ehi_ref, elo_ref, wr_s, ww_s):
    x = x_ref[...]

    @pl.when(pl.program_id(0) == 0)
    def _():
        e = vm_ref[0, 0] * emb_ref[...]
        hi = e.astype(jnp.bfloat16)
        ehi_ref[...] = hi
        elo_ref[...] = (e - hi.astype(jnp.float32)).astype(jnp.bfloat16)
        wr_s[...] = wrt_ref[...].T
        ww_s[...] = wwt_ref[...].T

    def amax(l):
        m = jnp.max(l, axis=-1, keepdims=True)
        ii = lax.broadcasted_iota(jnp.int32, l.shape, 1)
        return jnp.min(jnp.where(l == m, ii, l.shape[1]), axis=-1,
                       keepdims=True).astype(jnp.int32)

    rl = jnp.dot(x, wr_s[...], preferred_element_type=jnp.float32) \
        + br_ref[...]
    wl = jnp.dot(x, ww_s[...], preferred_element_type=jnp.float32) \
        + bw_ref[...]
    vl = jnp.dot(x, wv_ref[...], preferred_element_type=jnp.float32) \
        + bv_ref[...]
    ro_ref[...] = rl
    wo_ref[...] = wl
    vo_ref[...] = vl
    ridx = amax(rl)
    widx_ref[...] = amax(wl)
    wval_ref[...] = amax(vl)

    regs = regsT_ref[...].T
    col = lax.broadcasted_iota(jnp.int32, regs.shape, 1)
    sel = jnp.where(col == ridx, regs, 0).astype(jnp.bfloat16)
    ones_col = jnp.ones((_NREG, 1), jnp.bfloat16)
    rvf = jnp.dot(sel, ones_col, preferred_element_type=jnp.float32)
    rv = rvf.astype(jnp.int32)
    rv_ref[...] = rv

    rvc = jnp.minimum(jnp.maximum(rv, 0), _VR - 1)
    vcol = lax.broadcasted_iota(jnp.int32, (rv.shape[0], _VR), 1)
    onehot = (vcol == rvc).astype(jnp.bfloat16)
    fb_ref[...] = (
        jnp.dot(onehot, ehi_ref[...], preferred_element_type=jnp.float32)
        + jnp.dot(onehot, elo_ref[...], preferred_element_type=jnp.float32))


def _tc_call(x, registers, w_r_t, b_r, w_w_t, b_w, w_v, b_v, emb, vm):
    f32 = jnp.float32
    i32 = jnp.int32
    in_specs = [
        pl.BlockSpec((_BM, _D), lambda i: (i, 0)),
        pl.BlockSpec((_NREG, _BM), lambda i: (0, i)),
        pl.BlockSpec((_NREG + 1, _D), lambda i: (0, 0)),
        pl.BlockSpec((1, _NREG + 1), lambda i: (0, 0)),
        pl.BlockSpec((_NREG + 1, _D), lambda i: (0, 0)),
        pl.BlockSpec((1, _NREG + 1), lambda i: (0, 0)),
        pl.BlockSpec((_D, _VR), lambda i: (0, 0)),
        pl.BlockSpec((1, _VR), lambda i: (0, 0)),
        pl.BlockSpec((_VR, _D), lambda i: (0, 0)),
        pl.BlockSpec((1, 1), lambda i: (0, 0)),
    ]
    out_specs = [
        pl.BlockSpec((_BM, _NREG + 1), lambda i: (i, 0)),
        pl.BlockSpec((_BM, _NREG + 1), lambda i: (i, 0)),
        pl.BlockSpec((_BM, _VR), lambda i: (i, 0)),
        pl.BlockSpec((_BM, 1), lambda i: (i, 0)),
        pl.BlockSpec((_BM, 1), lambda i: (i, 0)),
        pl.BlockSpec((_BM, 1), lambda i: (i, 0)),
        pl.BlockSpec((_BM, _D), lambda i: (i, 0)),
    ]
    out_shape = [
        jax.ShapeDtypeStruct((_B, _NREG + 1), f32),
        jax.ShapeDtypeStruct((_B, _NREG + 1), f32),
        jax.ShapeDtypeStruct((_B, _VR), f32),
        jax.ShapeDtypeStruct((_B, 1), i32),
        jax.ShapeDtypeStruct((_B, 1), i32),
        jax.ShapeDtypeStruct((_B, 1), i32),
        jax.ShapeDtypeStruct((_B, _D), f32),
    ]
    return pl.pallas_call(
        _tc_body,
        grid=(_G,),
        in_specs=in_specs,
        out_specs=out_specs,
        out_shape=out_shape,
        scratch_shapes=[
            pltpu.VMEM((_VR, _D), jnp.bfloat16),
            pltpu.VMEM((_VR, _D), jnp.bfloat16),
            pltpu.VMEM((_D, _NREG + 1), jnp.float32),
            pltpu.VMEM((_D, _NREG + 1), jnp.float32),
        ],
        compiler_params=pltpu.CompilerParams(
            dimension_semantics=("arbitrary",)),
    )(x, registers, w_r_t, b_r, w_w_t, b_w, w_v, b_v, emb, vm)


def _sc_body(regsT_hbm, widx_hbm, wval_hbm, rv_hbm,
             nregsT_hbm, rvflat_hbm,
             widx_v, wval_v, rv_v, regsT_v, rvf_v, sem_idx, sem_regs):
    wid = lax.axis_index("s") * _NC + lax.axis_index("c")
    base = wid * _RPW
    c_wi = pltpu.async_copy(widx_hbm.at[pl.ds(base, _RPW)], widx_v, sem_idx)
    c_wv = pltpu.async_copy(wval_hbm.at[pl.ds(base, _RPW)], wval_v, sem_idx)
    c_rv = pltpu.async_copy(rv_hbm.at[pl.ds(base, _RPW)], rv_v, sem_idx)
    c_rg = pltpu.async_copy(
        regsT_hbm.at[:, pl.ds(base, _RPW)], regsT_v, sem_regs)
    c_wi.wait()
    c_wv.wait()
    c_rv.wait()
    c_rg.wait()

    zeros16 = jnp.zeros((_L,), jnp.int32)
    for g in range(_GRP):
        rows16 = lax.iota(jnp.int32, _L) + (g * _L)
        wi = plsc.load_gather(widx_v, [rows16, zeros16])
        wv = plsc.load_gather(wval_v, [rows16, zeros16])
        rvg = plsc.load_gather(rv_v, [rows16, zeros16])
        rvf_v[pl.ds(g * _L, _L)] = rvg
        wmask = wi < _NREG
        wcol = jnp.minimum(wi, _NREG - 1)
        plsc.store_scatter(regsT_v, [wcol, rows16], wv, mask=wmask)

    pltpu.sync_copy(regsT_v, nregsT_hbm.at[:, pl.ds(base, _RPW)])
    pltpu.sync_copy(rvf_v, rvflat_hbm.at[pl.ds(base, _RPW)])


def _sc_call(regsT, widx, wval, rv):
    i32 = jnp.int32
    mesh = plsc.VectorSubcoreMesh(core_axis_name="c", subcore_axis_name="s")
    cp = pltpu.CompilerParams()
    if "needs_layout_passes" in pltpu.CompilerParams.__dataclass_fields__:
        cp = dataclasses.replace(cp, needs_layout_passes=False)
    kern = pl.kernel(
        _sc_body,
        out_type=[
            jax.ShapeDtypeStruct((_NREG, _B), i32),
            jax.ShapeDtypeStruct((_B,), i32),
        ],
        mesh=mesh,
        scratch_types=[
            pltpu.VMEM((_RPW, 1), i32),
            pltpu.VMEM((_RPW, 1), i32),
            pltpu.VMEM((_RPW, 1), i32),
            pltpu.VMEM((_NREG, _RPW), i32),
            pltpu.VMEM((_RPW,), i32),
            pltpu.SemaphoreType.DMA,
            pltpu.SemaphoreType.DMA,
        ],
        compiler_params=cp,
    )
    return kern(regsT, widx, wval, rv)


def kernel(x, registers, W_read, b_read, W_write, b_write, W_val, b_val,
           value_emb, value_mix):
    br = b_read.reshape(1, _NREG + 1)
    bw = b_write.reshape(1, _NREG + 1)
    bv = b_val.reshape(1, _VR)
    vm = value_mix.reshape(1, 1)
    regsT = registers.T
    ro, wo, vo, widx, wval, rv, fb = _tc_call(
        x, regsT, W_read.T, br, W_write.T, bw, W_val, bv, value_emb, vm)
    nregsT, rvflat = _sc_call(regsT, widx, wval, rv)
    return (ro, wo, vo, nregsT.T, rvflat, fb)


# speedup vs baseline: 1.4304x; 1.0910x over previous
import dataclasses

import jax
import jax.numpy as jnp
from jax import lax
from jax.experimental import pallas as pl
from jax.experimental.pallas import tpu as pltpu
from jax.experimental.pallas import tpu_sc as plsc

_B = 4096
_D = 2048
_NREG = 64
_VR = 256

_BM = 512
_G = _B // _BM

_NC = 2
_NS = 16
_NW = _NC * _NS
_RPW = _B // _NW
_L = 16
_GRP = _RPW // _L


def _tc_body(x_ref, regsT_ref, wrt_ref, br_ref, wwt_ref, bw_ref, wv_ref,
             bv_ref, emb_ref, vm_ref,
             ro_ref, wo_ref, vo_ref, idx3_ref, fb_ref,
             ehilo_ref, wr_s, ww_s):
    x = x_ref[...]

    @pl.when(pl.program_id(0) == 0)
    def _():
        e = vm_ref[0, 0] * emb_ref[...]
        hi = e.astype(jnp.bfloat16)
        ehilo_ref[0:_VR, :] = hi
        ehilo_ref[_VR:, :] = (e - hi.astype(jnp.float32)).astype(jnp.bfloat16)
        wr_s[...] = wrt_ref[...].T
        ww_s[...] = wwt_ref[...].T

    def amax(l):
        m = jnp.max(l, axis=-1, keepdims=True)
        ii = lax.broadcasted_iota(jnp.int32, l.shape, 1)
        return jnp.min(jnp.where(l == m, ii, l.shape[1]), axis=-1,
                       keepdims=True).astype(jnp.int32)

    rl = jnp.dot(x, wr_s[...], preferred_element_type=jnp.float32) \
        + br_ref[...]
    wl = jnp.dot(x, ww_s[...], preferred_element_type=jnp.float32) \
        + bw_ref[...]
    vl = jnp.dot(x, wv_ref[...], preferred_element_type=jnp.float32) \
        + bv_ref[...]
    ro_ref[...] = rl
    wo_ref[...] = wl
    vo_ref[...] = vl
    ridx = amax(rl)
    widx = amax(wl)
    wval = amax(vl)

    regs = regsT_ref[...].T
    col = lax.broadcasted_iota(jnp.int32, regs.shape, 1)
    sel = jnp.where(col == ridx, regs, 0).astype(jnp.bfloat16)
    ones_col = jnp.ones((_NREG, 1), jnp.bfloat16)
    rvf = jnp.dot(sel, ones_col, preferred_element_type=jnp.float32)
    rv = rvf.astype(jnp.int32)
    idx3_ref[...] = jnp.concatenate([widx, wval, rv, rv], axis=1)

    rvc = jnp.minimum(jnp.maximum(rv, 0), _VR - 1)
    vcol = lax.broadcasted_iota(jnp.int32, (rv.shape[0], 2 * _VR), 1)
    onehot2 = ((vcol & (_VR - 1)) == rvc).astype(jnp.bfloat16)
    fb_ref[...] = jnp.dot(onehot2, ehilo_ref[...],
                          preferred_element_type=jnp.float32)


def _tc_call(x, registers, w_r_t, b_r, w_w_t, b_w, w_v, b_v, emb, vm):
    f32 = jnp.float32
    i32 = jnp.int32
    in_specs = [
        pl.BlockSpec((_BM, _D), lambda i: (i, 0)),
        pl.BlockSpec((_NREG, _BM), lambda i: (0, i)),
        pl.BlockSpec((_NREG + 1, _D), lambda i: (0, 0)),
        pl.BlockSpec((1, _NREG + 1), lambda i: (0, 0)),
        pl.BlockSpec((_NREG + 1, _D), lambda i: (0, 0)),
        pl.BlockSpec((1, _NREG + 1), lambda i: (0, 0)),
        pl.BlockSpec((_D, _VR), lambda i: (0, 0)),
        pl.BlockSpec((1, _VR), lambda i: (0, 0)),
        pl.BlockSpec((_VR, _D), lambda i: (0, 0)),
        pl.BlockSpec((1, 1), lambda i: (0, 0)),
    ]
    out_specs = [
        pl.BlockSpec((_BM, _NREG + 1), lambda i: (i, 0)),
        pl.BlockSpec((_BM, _NREG + 1), lambda i: (i, 0)),
        pl.BlockSpec((_BM, _VR), lambda i: (i, 0)),
        pl.BlockSpec((_BM, 4), lambda i: (i, 0)),
        pl.BlockSpec((_BM, _D), lambda i: (i, 0)),
    ]
    out_shape = [
        jax.ShapeDtypeStruct((_B, _NREG + 1), f32),
        jax.ShapeDtypeStruct((_B, _NREG + 1), f32),
        jax.ShapeDtypeStruct((_B, _VR), f32),
        jax.ShapeDtypeStruct((_B, 4), i32),
        jax.ShapeDtypeStruct((_B, _D), f32),
    ]
    return pl.pallas_call(
        _tc_body,
        grid=(_G,),
        in_specs=in_specs,
        out_specs=out_specs,
        out_shape=out_shape,
        scratch_shapes=[
            pltpu.VMEM((2 * _VR, _D), jnp.bfloat16),
            pltpu.VMEM((_D, _NREG + 1), jnp.float32),
            pltpu.VMEM((_D, _NREG + 1), jnp.float32),
        ],
        compiler_params=pltpu.CompilerParams(
            dimension_semantics=("arbitrary",)),
    )(x, registers, w_r_t, b_r, w_w_t, b_w, w_v, b_v, emb, vm)


def _sc_body(regsT_hbm, idx3_hbm,
             nregsT_hbm, rvflat_hbm,
             idx3_v, regsT_v, rvf_v, sem_idx, sem_regs):
    wid = lax.axis_index("s") * _NC + lax.axis_index("c")
    base = wid * _RPW
    c_ix = pltpu.async_copy(idx3_hbm.at[pl.ds(base, _RPW)], idx3_v, sem_idx)
    c_rg = pltpu.async_copy(
        regsT_hbm.at[:, pl.ds(base, _RPW)], regsT_v, sem_regs)
    c_ix.wait()
    c_rg.wait()

    zeros16 = jnp.zeros((_L,), jnp.int32)
    for g in range(_GRP):
        rows16 = lax.iota(jnp.int32, _L) + (g * _L)
        wi = plsc.load_gather(idx3_v, [rows16, zeros16])
        wv = plsc.load_gather(idx3_v, [rows16, zeros16 + 1])
        rvg = plsc.load_gather(idx3_v, [rows16, zeros16 + 2])
        rvf_v[pl.ds(g * _L, _L)] = rvg
        wmask = wi < _NREG
        wcol = jnp.minimum(wi, _NREG - 1)
        plsc.store_scatter(regsT_v, [wcol, rows16], wv, mask=wmask)

    pltpu.sync_copy(regsT_v, nregsT_hbm.at[:, pl.ds(base, _RPW)])
    pltpu.sync_copy(rvf_v, rvflat_hbm.at[pl.ds(base, _RPW)])


def _sc_call(regsT, idx3):
    i32 = jnp.int32
    mesh = plsc.VectorSubcoreMesh(core_axis_name="c", subcore_axis_name="s")
    cp = pltpu.CompilerParams()
    if "needs_layout_passes" in pltpu.CompilerParams.__dataclass_fields__:
        cp = dataclasses.replace(cp, needs_layout_passes=False)
    kern = pl.kernel(
        _sc_body,
        out_type=[
            jax.ShapeDtypeStruct((_NREG, _B), i32),
            jax.ShapeDtypeStruct((_B,), i32),
        ],
        mesh=mesh,
        scratch_types=[
            pltpu.VMEM((_RPW, 4), i32),
            pltpu.VMEM((_NREG, _RPW), i32),
            pltpu.VMEM((_RPW,), i32),
            pltpu.SemaphoreType.DMA,
            pltpu.SemaphoreType.DMA,
        ],
        compiler_params=cp,
    )
    return kern(regsT, idx3)


def kernel(x, registers, W_read, b_read, W_write, b_write, W_val, b_val,
           value_emb, value_mix):
    br = b_read.reshape(1, _NREG + 1)
    bw = b_write.reshape(1, _NREG + 1)
    bv = b_val.reshape(1, _VR)
    vm = value_mix.reshape(1, 1)
    regsT = registers.T
    ro, wo, vo, idx3, fb = _tc_call(
        x, regsT, W_read.T, br, W_write.T, bw, W_val, bv, value_emb, vm)
    nregsT, rvflat = _sc_call(regsT, idx3)
    return (ro, wo, vo, nregsT.T, rvflat, fb)


# speedup vs baseline: 1.4348x; 1.0031x over previous
import dataclasses

import jax
import jax.numpy as jnp
from jax import lax
from jax.experimental import pallas as pl
from jax.experimental.pallas import tpu as pltpu
from jax.experimental.pallas import tpu_sc as plsc

_B = 4096
_D = 2048
_NREG = 64
_VR = 256

_BM = 512
_G = _B // _BM

_NC = 2
_NS = 16
_NW = _NC * _NS
_RPW = _B // _NW
_L = 16
_GRP = _RPW // _L


def _tc_body(x_ref, regsT_ref, wrt_ref, br_ref, wwt_ref, bw_ref, wv_ref,
             bv_ref, emb_ref, vm_ref,
             ro_ref, wo_ref, vo_ref, idx3_ref, fb_ref,
             ehilo_ref, wr_s, ww_s):
    x = x_ref[...]

    @pl.when(pl.program_id(0) == 0)
    def _():
        e = vm_ref[0, 0] * emb_ref[...]
        hi = e.astype(jnp.bfloat16)
        ehilo_ref[0:_VR, :] = hi
        ehilo_ref[_VR:, :] = (e - hi.astype(jnp.float32)).astype(jnp.bfloat16)
        wr_s[...] = wrt_ref[...].T
        ww_s[...] = wwt_ref[...].T

    def amax(l):
        m = jnp.max(l, axis=-1, keepdims=True)
        ii = lax.broadcasted_iota(jnp.int32, l.shape, 1)
        return jnp.min(jnp.where(l == m, ii, l.shape[1]), axis=-1,
                       keepdims=True).astype(jnp.int32)

    rl = jnp.dot(x, wr_s[...], preferred_element_type=jnp.float32) \
        + br_ref[...]
    wl = jnp.dot(x, ww_s[...], preferred_element_type=jnp.float32) \
        + bw_ref[...]
    vl = jnp.dot(x, wv_ref[...], preferred_element_type=jnp.float32) \
        + bv_ref[...]
    ro_ref[...] = rl
    wo_ref[...] = wl
    vo_ref[...] = vl
    ridx = amax(rl)
    widx = amax(wl)
    wval = amax(vl)

    regs = regsT_ref[...].T
    col = lax.broadcasted_iota(jnp.int32, regs.shape, 1)
    sel = jnp.where(col == ridx, regs, 0).astype(jnp.bfloat16)
    ones_col = jnp.ones((_NREG, 1), jnp.bfloat16)
    rvf = jnp.dot(sel, ones_col, preferred_element_type=jnp.float32)
    rv = rvf.astype(jnp.int32)
    idx3_ref[...] = jnp.concatenate([widx, wval, rv, rv], axis=1)

    rvc = jnp.minimum(jnp.maximum(rv, 0), _VR - 1)
    vcol = lax.broadcasted_iota(jnp.int32, (rv.shape[0], 2 * _VR), 1)
    onehot2 = ((vcol & (_VR - 1)) == rvc).astype(jnp.bfloat16)
    fb_ref[...] = jnp.dot(onehot2, ehilo_ref[...],
                          preferred_element_type=jnp.float32)


def _tc_call(x, registers, w_r_t, b_r, w_w_t, b_w, w_v, b_v, emb, vm):
    f32 = jnp.float32
    i32 = jnp.int32
    in_specs = [
        pl.BlockSpec((_BM, _D), lambda i: (i, 0)),
        pl.BlockSpec((_NREG, _BM), lambda i: (0, i)),
        pl.BlockSpec((_NREG + 1, _D), lambda i: (0, 0)),
        pl.BlockSpec((1, _NREG + 1), lambda i: (0, 0)),
        pl.BlockSpec((_NREG + 1, _D), lambda i: (0, 0)),
        pl.BlockSpec((1, _NREG + 1), lambda i: (0, 0)),
        pl.BlockSpec((_D, _VR), lambda i: (0, 0)),
        pl.BlockSpec((1, _VR), lambda i: (0, 0)),
        pl.BlockSpec((_VR, _D), lambda i: (0, 0)),
        pl.BlockSpec((1, 1), lambda i: (0, 0)),
    ]
    out_specs = [
        pl.BlockSpec((_BM, _NREG + 1), lambda i: (i, 0)),
        pl.BlockSpec((_BM, _NREG + 1), lambda i: (i, 0)),
        pl.BlockSpec((_BM, _VR), lambda i: (i, 0)),
        pl.BlockSpec((_BM, 4), lambda i: (i, 0)),
        pl.BlockSpec((_BM, _D), lambda i: (i, 0)),
    ]
    out_shape = [
        jax.ShapeDtypeStruct((_B, _NREG + 1), f32),
        jax.ShapeDtypeStruct((_B, _NREG + 1), f32),
        jax.ShapeDtypeStruct((_B, _VR), f32),
        jax.ShapeDtypeStruct((_B, 4), i32),
        jax.ShapeDtypeStruct((_B, _D), f32),
    ]
    return pl.pallas_call(
        _tc_body,
        grid=(_G,),
        in_specs=in_specs,
        out_specs=out_specs,
        out_shape=out_shape,
        scratch_shapes=[
            pltpu.VMEM((2 * _VR, _D), jnp.bfloat16),
            pltpu.VMEM((_D, _NREG + 1), jnp.float32),
            pltpu.VMEM((_D, _NREG + 1), jnp.float32),
        ],
        compiler_params=pltpu.CompilerParams(
            dimension_semantics=("arbitrary",)),
    )(x, registers, w_r_t, b_r, w_w_t, b_w, w_v, b_v, emb, vm)


def _sc_body(regsT_hbm, idx3_hbm,
             nregsT_hbm, rvflat_hbm,
             idx3_v, regsT_v, rvf_v, sem_idx, sem_regs):
    wid = lax.axis_index("s") * _NC + lax.axis_index("c")
    base = wid * _RPW
    c_ix = pltpu.async_copy(idx3_hbm.at[pl.ds(base, _RPW)], idx3_v, sem_idx)
    c_rg = pltpu.async_copy(
        regsT_hbm.at[:, pl.ds(base, _RPW)], regsT_v, sem_regs)
    c_ix.wait()
    c_rg.wait()

    zeros16 = jnp.zeros((_L,), jnp.int32)

    @pl.loop(0, _GRP)
    def _(g):
        rows16 = lax.iota(jnp.int32, _L) + (g * _L)
        wi = plsc.load_gather(idx3_v, [rows16, zeros16])
        wv = plsc.load_gather(idx3_v, [rows16, zeros16 + 1])
        rvg = plsc.load_gather(idx3_v, [rows16, zeros16 + 2])
        rvf_v[pl.ds(g * _L, _L)] = rvg
        wmask = wi < _NREG
        wcol = jnp.minimum(wi, _NREG - 1)
        plsc.store_scatter(regsT_v, [wcol, rows16], wv, mask=wmask)

    pltpu.sync_copy(regsT_v, nregsT_hbm.at[:, pl.ds(base, _RPW)])
    pltpu.sync_copy(rvf_v, rvflat_hbm.at[pl.ds(base, _RPW)])


def _sc_call(regsT, idx3):
    i32 = jnp.int32
    mesh = plsc.VectorSubcoreMesh(core_axis_name="c", subcore_axis_name="s")
    cp = pltpu.CompilerParams()
    if "needs_layout_passes" in pltpu.CompilerParams.__dataclass_fields__:
        cp = dataclasses.replace(cp, needs_layout_passes=False)
    kern = pl.kernel(
        _sc_body,
        out_type=[
            jax.ShapeDtypeStruct((_NREG, _B), i32),
            jax.ShapeDtypeStruct((_B,), i32),
        ],
        mesh=mesh,
        scratch_types=[
            pltpu.VMEM((_RPW, 4), i32),
            pltpu.VMEM((_NREG, _RPW), i32),
            pltpu.VMEM((_RPW,), i32),
            pltpu.SemaphoreType.DMA,
            pltpu.SemaphoreType.DMA,
        ],
        compiler_params=cp,
    )
    return kern(regsT, idx3)


def kernel(x, registers, W_read, b_read, W_write, b_write, W_val, b_val,
           value_emb, value_mix):
    br = b_read.reshape(1, _NREG + 1)
    bw = b_write.reshape(1, _NREG + 1)
    bv = b_val.reshape(1, _VR)
    vm = value_mix.reshape(1, 1)
    regsT = registers.T
    ro, wo, vo, idx3, fb = _tc_call(
        x, regsT, W_read.T, br, W_write.T, bw, W_val, bv, value_emb, vm)
    nregsT, rvflat = _sc_call(regsT, idx3)
    return (ro, wo, vo, nregsT.T, rvflat, fb)


# speedup vs baseline: 1.4607x; 1.0180x over previous
import dataclasses

import jax
import jax.numpy as jnp
from jax import lax
from jax.experimental import pallas as pl
from jax.experimental.pallas import tpu as pltpu
from jax.experimental.pallas import tpu_sc as plsc

_B = 4096
_D = 2048
_NREG = 64
_VR = 256

_BM = 512
_G = _B // _BM

_NC = 2
_NS = 16
_NW = _NC * _NS
_RPW = _B // _NW
_L = 16
_GRP = _RPW // _L


def _tc_body(x_ref, regsT_ref, wrt_ref, br_ref, wwt_ref, bw_ref, wv_ref,
             bv_ref, emb_ref, vm_ref,
             ro_ref, wo_ref, vo_ref, idx3_ref, fb_ref,
             ehilo_ref, wr_s, ww_s):
    x = x_ref[...]

    @pl.when(pl.program_id(0) == 0)
    def _():
        e = vm_ref[0, 0] * emb_ref[...]
        hi = e.astype(jnp.bfloat16)
        ehilo_ref[0:_VR, :] = hi
        ehilo_ref[_VR:, :] = (e - hi.astype(jnp.float32)).astype(jnp.bfloat16)
        wr_s[...] = wrt_ref[...].T
        ww_s[...] = wwt_ref[...].T

    def amax(l):
        m = jnp.max(l, axis=-1, keepdims=True)
        ii = lax.broadcasted_iota(jnp.int32, l.shape, 1)
        return jnp.min(jnp.where(l == m, ii, l.shape[1]), axis=-1,
                       keepdims=True).astype(jnp.int32)

    rl = jnp.dot(x, wr_s[...], preferred_element_type=jnp.float32) \
        + br_ref[...]
    wl = jnp.dot(x, ww_s[...], preferred_element_type=jnp.float32) \
        + bw_ref[...]
    vl = jnp.dot(x, wv_ref[...], preferred_element_type=jnp.float32) \
        + bv_ref[...]
    ro_ref[...] = rl.T
    wo_ref[...] = wl.T
    vo_ref[...] = vl
    ridx = amax(rl)
    widx = amax(wl)
    wval = amax(vl)

    regs = regsT_ref[...].T
    col = lax.broadcasted_iota(jnp.int32, regs.shape, 1)
    sel = jnp.where(col == ridx, regs, 0).astype(jnp.bfloat16)
    ones_col = jnp.ones((_NREG, 1), jnp.bfloat16)
    rvf = jnp.dot(sel, ones_col, preferred_element_type=jnp.float32)
    rv = rvf.astype(jnp.int32)
    idx3_ref[...] = jnp.concatenate([widx, wval, rv, rv], axis=1)

    rvc = jnp.minimum(jnp.maximum(rv, 0), _VR - 1)
    vcol = lax.broadcasted_iota(jnp.int32, (rv.shape[0], 2 * _VR), 1)
    onehot2 = ((vcol & (_VR - 1)) == rvc).astype(jnp.bfloat16)
    fb_ref[...] = jnp.dot(onehot2, ehilo_ref[...],
                          preferred_element_type=jnp.float32)


def _tc_call(x, registers, w_r_t, b_r, w_w_t, b_w, w_v, b_v, emb, vm):
    f32 = jnp.float32
    i32 = jnp.int32
    in_specs = [
        pl.BlockSpec((_BM, _D), lambda i: (i, 0)),
        pl.BlockSpec((_NREG, _BM), lambda i: (0, i)),
        pl.BlockSpec((_NREG + 1, _D), lambda i: (0, 0)),
        pl.BlockSpec((1, _NREG + 1), lambda i: (0, 0)),
        pl.BlockSpec((_NREG + 1, _D), lambda i: (0, 0)),
        pl.BlockSpec((1, _NREG + 1), lambda i: (0, 0)),
        pl.BlockSpec((_D, _VR), lambda i: (0, 0)),
        pl.BlockSpec((1, _VR), lambda i: (0, 0)),
        pl.BlockSpec((_VR, _D), lambda i: (0, 0)),
        pl.BlockSpec((1, 1), lambda i: (0, 0)),
    ]
    out_specs = [
        pl.BlockSpec((_NREG + 1, _BM), lambda i: (0, i)),
        pl.BlockSpec((_NREG + 1, _BM), lambda i: (0, i)),
        pl.BlockSpec((_BM, _VR), lambda i: (i, 0)),
        pl.BlockSpec((_BM, 4), lambda i: (i, 0)),
        pl.BlockSpec((_BM, _D), lambda i: (i, 0)),
    ]
    out_shape = [
        jax.ShapeDtypeStruct((_NREG + 1, _B), f32),
        jax.ShapeDtypeStruct((_NREG + 1, _B), f32),
        jax.ShapeDtypeStruct((_B, _VR), f32),
        jax.ShapeDtypeStruct((_B, 4), i32),
        jax.ShapeDtypeStruct((_B, _D), f32),
    ]
    return pl.pallas_call(
        _tc_body,
        grid=(_G,),
        in_specs=in_specs,
        out_specs=out_specs,
        out_shape=out_shape,
        scratch_shapes=[
            pltpu.VMEM((2 * _VR, _D), jnp.bfloat16),
            pltpu.VMEM((_D, _NREG + 1), jnp.float32),
            pltpu.VMEM((_D, _NREG + 1), jnp.float32),
        ],
        compiler_params=pltpu.CompilerParams(
            dimension_semantics=("arbitrary",)),
    )(x, registers, w_r_t, b_r, w_w_t, b_w, w_v, b_v, emb, vm)


def _sc_body(regsT_hbm, idx3_hbm,
             nregsT_hbm, rvflat_hbm,
             idx3_v, regsT_v, rvf_v, sem_idx, sem_regs):
    wid = lax.axis_index("s") * _NC + lax.axis_index("c")
    base = wid * _RPW
    c_ix = pltpu.async_copy(idx3_hbm.at[pl.ds(base, _RPW)], idx3_v, sem_idx)
    c_rg = pltpu.async_copy(
        regsT_hbm.at[:, pl.ds(base, _RPW)], regsT_v, sem_regs)
    c_ix.wait()
    c_rg.wait()

    zeros16 = jnp.zeros((_L,), jnp.int32)

    @pl.loop(0, _GRP)
    def _(g):
        rows16 = lax.iota(jnp.int32, _L) + (g * _L)
        wi = plsc.load_gather(idx3_v, [rows16, zeros16])
        wv = plsc.load_gather(idx3_v, [rows16, zeros16 + 1])
        rvg = plsc.load_gather(idx3_v, [rows16, zeros16 + 2])
        rvf_v[pl.ds(g * _L, _L)] = rvg
        wmask = wi < _NREG
        wcol = jnp.minimum(wi, _NREG - 1)
        plsc.store_scatter(regsT_v, [wcol, rows16], wv, mask=wmask)

    pltpu.sync_copy(regsT_v, nregsT_hbm.at[:, pl.ds(base, _RPW)])
    pltpu.sync_copy(rvf_v, rvflat_hbm.at[pl.ds(base, _RPW)])


def _sc_call(regsT, idx3):
    i32 = jnp.int32
    mesh = plsc.VectorSubcoreMesh(core_axis_name="c", subcore_axis_name="s")
    cp = pltpu.CompilerParams()
    if "needs_layout_passes" in pltpu.CompilerParams.__dataclass_fields__:
        cp = dataclasses.replace(cp, needs_layout_passes=False)
    kern = pl.kernel(
        _sc_body,
        out_type=[
            jax.ShapeDtypeStruct((_NREG, _B), i32),
            jax.ShapeDtypeStruct((_B,), i32),
        ],
        mesh=mesh,
        scratch_types=[
            pltpu.VMEM((_RPW, 4), i32),
            pltpu.VMEM((_NREG, _RPW), i32),
            pltpu.VMEM((_RPW,), i32),
            pltpu.SemaphoreType.DMA,
            pltpu.SemaphoreType.DMA,
        ],
        compiler_params=cp,
    )
    return kern(regsT, idx3)


def kernel(x, registers, W_read, b_read, W_write, b_write, W_val, b_val,
           value_emb, value_mix):
    br = b_read.reshape(1, _NREG + 1)
    bw = b_write.reshape(1, _NREG + 1)
    bv = b_val.reshape(1, _VR)
    vm = value_mix.reshape(1, 1)
    regsT = registers.T
    roT, woT, vo, idx3, fb = _tc_call(
        x, regsT, W_read.T, br, W_write.T, bw, W_val, bv, value_emb, vm)
    nregsT, rvflat = _sc_call(regsT, idx3)
    return (roT.T, woT.T, vo, nregsT.T, rvflat, fb)
